# baseline (speedup 1.0000x reference)
import jax
import jax.numpy as jnp
from jax import lax
from jax.experimental import pallas as pl
from jax.experimental.pallas import tpu as pltpu
from jax.experimental.pallas import tpu_sc as plsc

HASH_SZ_MASK = 65536 - 1
MULT = 1000003
B, T = 4, 8192
D = 1000
V = 65536
N_TOK = B * T
NW = 32
LANES = 16
PAD = 8
IDS_CHUNK = 2048
ROWS_PER_W = (D + NW - 1) // NW


def _sc_kernel(ids_hbm, table_hbm, out_hbm, ids_v, keys_v, row_v,
               stage_a, stage_b, rsem, rsem2, osem_a, osem_b):
    wid = lax.axis_index("s") * 2 + lax.axis_index("c")

    for cc in range(N_TOK // IDS_CHUNK):
        base = cc * IDS_CHUNK
        if cc % (T // IDS_CHUNK) == 0:
            ids_v[pl.ds(0, LANES)] = jnp.zeros((LANES,), jnp.int32)
        else:
            pltpu.sync_copy(ids_hbm.at[pl.ds(base - PAD, PAD)],
                            ids_v.at[pl.ds(0, PAD)])
        pltpu.sync_copy(ids_hbm.at[pl.ds(base, IDS_CHUNK)],
                        ids_v.at[pl.ds(PAD, IDS_CHUNK)])

        @plsc.parallel_loop(0, IDS_CHUNK // LANES, unroll=4)
        def _(i):
            curr = ids_v[pl.ds(PAD + i * LANES, LANES)]
            prev = ids_v[pl.ds(PAD - 1 + i * LANES, LANES)]
            key = (prev * MULT + curr) & HASH_SZ_MASK
            keys_v[pl.ds(base + i * LANES, LANES)] = key

    stages = (stage_a, stage_b)
    osems = (osem_a, osem_b)

    def do_row(k, _):
        c = wid + k * NW

        @pl.when(c < D)
        def _():
            j = c >> 3
            s = c & 7
            w1 = pltpu.async_copy(
                table_hbm.at[j, pl.ds(0, 256), s, :],
                row_v.at[pl.ds(0, 256), :], rsem)
            w2 = pltpu.async_copy(
                table_hbm.at[j, pl.ds(256, 256), s, :],
                row_v.at[pl.ds(256, 256), :], rsem2)
            w1.wait()
            w2.wait()
            for b in range(B):
                nb = b % 2
                stage = stages[nb]
                if b >= 2:
                    pltpu.make_async_copy(
                        stage, out_hbm.at[b, c, :], osems[nb]).wait()
                else:
                    @pl.when(k > 0)
                    def _():
                        pltpu.make_async_copy(
                            stage, out_hbm.at[b, c, :], osems[nb]).wait()

                @plsc.parallel_loop(0, T // LANES, unroll=16)
                def _(g):
                    k16 = keys_v[pl.ds(b * T + g * LANES, LANES)]
                    stage[pl.ds(g * LANES, LANES)] = plsc.load_gather(
                        row_v, [k16 >> 7, k16 & 127])

                pltpu.async_copy(stage, out_hbm.at[b, c, :], osems[nb])
        return 0

    lax.fori_loop(0, ROWS_PER_W, do_row, 0)
    for nb in range(2):
        pltpu.make_async_copy(
            stages[nb], out_hbm.at[2 + nb, 0, :], osems[nb]).wait()


def kernel(input_ids, embedding_weight):
    ids_flat = input_ids.reshape(N_TOK).astype(jnp.int32)
    table_t = embedding_weight.reshape(V // 128, 128, D // 8, 8).transpose(
        2, 0, 3, 1)
    mesh = plsc.VectorSubcoreMesh(core_axis_name="c", subcore_axis_name="s")
    out = pl.kernel(
        _sc_kernel,
        mesh=mesh,
        compiler_params=pltpu.CompilerParams(
            use_tc_tiling_on_sc=False, needs_layout_passes=False),
        out_type=jax.ShapeDtypeStruct((B, D, T), jnp.float32),
        scratch_types=[
            pltpu.VMEM((PAD + IDS_CHUNK,), jnp.int32),
            pltpu.VMEM((N_TOK,), jnp.int32),
            pltpu.VMEM((V // 128, 128), jnp.float32),
            pltpu.VMEM((T,), jnp.float32),
            pltpu.VMEM((T,), jnp.float32),
            pltpu.SemaphoreType.DMA,
            pltpu.SemaphoreType.DMA,
            pltpu.SemaphoreType.DMA,
            pltpu.SemaphoreType.DMA,
        ],
    )(ids_flat, table_t)
    return out.transpose(0, 2, 1)


# speedup vs baseline: 1.0603x; 1.0603x over previous
import jax
import jax.numpy as jnp
from jax import lax
from jax.experimental import pallas as pl
from jax.experimental.pallas import tpu as pltpu
from jax.experimental.pallas import tpu_sc as plsc

HASH_SZ_MASK = 65536 - 1
MULT = 1000003
B, T = 4, 8192
D = 1000
V = 65536
N_TOK = B * T
NW = 32
LANES = 16
PAD = 8
IDS_CHUNK = 2048
ROWS_PER_W = (D + NW - 1) // NW


def _sc_kernel(ids_hbm, table_hbm, out_hbm, ids_v, keys_v, row_v,
               stage_a, stage_b, rsem, rsem2, osem_a, osem_b):
    wid = lax.axis_index("s") * 2 + lax.axis_index("c")

    for cc in range(N_TOK // IDS_CHUNK):
        base = cc * IDS_CHUNK
        if cc % (T // IDS_CHUNK) == 0:
            ids_v[pl.ds(0, LANES)] = jnp.zeros((LANES,), jnp.int32)
        else:
            pltpu.sync_copy(ids_hbm.at[pl.ds(base - PAD, PAD)],
                            ids_v.at[pl.ds(0, PAD)])
        pltpu.sync_copy(ids_hbm.at[pl.ds(base, IDS_CHUNK)],
                        ids_v.at[pl.ds(PAD, IDS_CHUNK)])

        @plsc.parallel_loop(0, IDS_CHUNK // (2 * LANES), unroll=4)
        def _(i):
            curr0 = ids_v[pl.ds(PAD + i * 2 * LANES, LANES)]
            prev0 = ids_v[pl.ds(PAD - 1 + i * 2 * LANES, LANES)]
            curr1 = ids_v[pl.ds(PAD + i * 2 * LANES + LANES, LANES)]
            prev1 = ids_v[pl.ds(PAD - 1 + i * 2 * LANES + LANES, LANES)]
            key0 = (prev0 * MULT + curr0) & HASH_SZ_MASK
            key1 = (prev1 * MULT + curr1) & HASH_SZ_MASK
            keys_v[pl.ds(base + i * 2 * LANES, 2 * LANES)] = plsc.pack(
                key0, key1, format=plsc.PackFormat.INTERLEAVED)

    stages = (stage_a, stage_b)
    osems = (osem_a, osem_b)

    def do_row(k, _):
        c = wid + k * NW

        @pl.when(c < D)
        def _():
            j = c >> 3
            s = c & 7
            pltpu.async_copy(table_hbm.at[j, :, s, :], row_v, rsem).wait()
            for b in range(B):
                nb = b % 2
                stage = stages[nb]
                if b >= 2:
                    pltpu.make_async_copy(
                        stage, out_hbm.at[b, c, :], osems[nb]).wait()
                else:
                    @pl.when(k > 0)
                    def _():
                        pltpu.make_async_copy(
                            stage, out_hbm.at[b, c, :], osems[nb]).wait()

                @plsc.parallel_loop(0, T // (2 * LANES), unroll=8)
                def _(g):
                    pk = keys_v[pl.ds(b * T + g * 2 * LANES, 2 * LANES)]
                    k0, k1 = plsc.unpack(pk, format=plsc.PackFormat.INTERLEAVED)
                    k0 = k0 & HASH_SZ_MASK
                    k1 = k1 & HASH_SZ_MASK
                    stage[pl.ds(g * 2 * LANES, LANES)] = plsc.load_gather(
                        row_v, [k0 >> 7, k0 & 127])
                    stage[pl.ds(g * 2 * LANES + LANES, LANES)] = (
                        plsc.load_gather(row_v, [k1 >> 7, k1 & 127]))

                pltpu.async_copy(stage, out_hbm.at[b, c, :], osems[nb])
        return 0

    lax.fori_loop(0, ROWS_PER_W, do_row, 0)
    for nb in range(2):
        pltpu.make_async_copy(
            stages[nb], out_hbm.at[2 + nb, 0, :], osems[nb]).wait()


def kernel(input_ids, embedding_weight):
    ids_flat = input_ids.reshape(N_TOK).astype(jnp.int32)
    table_t = embedding_weight.reshape(V // 128, 128, D // 8, 8).transpose(
        2, 0, 3, 1)
    mesh = plsc.VectorSubcoreMesh(core_axis_name="c", subcore_axis_name="s")
    out = pl.kernel(
        _sc_kernel,
        mesh=mesh,
        compiler_params=pltpu.CompilerParams(
            use_tc_tiling_on_sc=False, needs_layout_passes=False),
        out_type=jax.ShapeDtypeStruct((B, D, T), jnp.float32),
        scratch_types=[
            pltpu.VMEM((PAD + IDS_CHUNK,), jnp.int32),
            pltpu.VMEM((N_TOK,), jnp.int16),
            pltpu.VMEM((V // 128, 128), jnp.float32),
            pltpu.VMEM((T,), jnp.float32),
            pltpu.VMEM((T,), jnp.float32),
            pltpu.SemaphoreType.DMA,
            pltpu.SemaphoreType.DMA,
            pltpu.SemaphoreType.DMA,
            pltpu.SemaphoreType.DMA,
        ],
    )(ids_flat, table_t)
    return out.transpose(0, 2, 1)
